# Initial kernel scaffold; baseline (speedup 1.0000x reference)
import jax
import jax.numpy as jnp
from jax.experimental import pallas as pl


def kernel(h, r, t, ent_emb, rel_emb):
    raise NotImplementedError("write your pallas kernel here")


# speedup vs baseline: 2.0305x; 2.0305x over previous
import functools

import jax
import jax.numpy as jnp
from jax import lax
from jax.experimental import pallas as pl
from jax.experimental.pallas import tpu as pltpu
from jax.experimental.pallas import tpu_sc as plsc

ENT_TOTAL = 100000
REL_TOTAL = 1000
EMB = 128
BATCH = 16384

L = 16
NC = 2
NS = 16
NW = NC * NS
BPW = BATCH // NW
CH = 128
NCH = BPW // CH


def _build():
  mesh = plsc.VectorSubcoreMesh(core_axis_name="c", subcore_axis_name="s")

  @functools.partial(
      pl.kernel,
      mesh=mesh,
      compiler_params=pltpu.CompilerParams(needs_layout_passes=False),
      out_type=jax.ShapeDtypeStruct((BATCH,), jnp.float32),
      scratch_types=[
          pltpu.VMEM((NCH, CH), jnp.int32),
          pltpu.VMEM((NCH, CH), jnp.int32),
          pltpu.VMEM((NCH, CH), jnp.int32),
          pltpu.VMEM((2, CH, EMB), jnp.float32),
          pltpu.VMEM((2, CH, EMB), jnp.float32),
          pltpu.VMEM((2, CH, EMB), jnp.float32),
          pltpu.VMEM((BPW,), jnp.float32),
          pltpu.VMEM((L, L + 1), jnp.float32),
          pltpu.SemaphoreType.DMA,
          pltpu.SemaphoreType.DMA,
      ],
  )
  def k(h_hbm, r_hbm, t_hbm, ent_hbm, rel_hbm, out_hbm,
        hidx, ridx, tidx, hrows, rrows, trows, outv, scr, sem0, sem1):
    wid = lax.axis_index("s") * NC + lax.axis_index("c")
    rowbase = wid * NCH
    pltpu.sync_copy(h_hbm.at[pl.ds(rowbase, NCH)], hidx)
    pltpu.sync_copy(r_hbm.at[pl.ds(rowbase, NCH)], ridx)
    pltpu.sync_copy(t_hbm.at[pl.ds(rowbase, NCH)], tidx)

    sems = (sem0, sem1)

    def fire(c, b):
      return (
          pltpu.async_copy(ent_hbm.at[hidx.at[c]], hrows.at[b], sems[b]),
          pltpu.async_copy(rel_hbm.at[ridx.at[c]], rrows.at[b], sems[b]),
          pltpu.async_copy(ent_hbm.at[tidx.at[c]], trows.at[b], sems[b]),
      )

    inflight = {0: fire(0, 0)}
    iota16 = lax.iota(jnp.int32, L)

    for c in range(NCH):
      b = c & 1
      if c + 1 < NCH:
        inflight[c + 1] = fire(c + 1, (c + 1) & 1)
      for cp in inflight.pop(c):
        cp.wait()
      hb, rb, tb = hrows.at[b], rrows.at[b], trows.at[b]

      def group(g, _, hb=hb, rb=rb, tb=tb, c=c):
        for j in range(L):
          i = g * L + j
          acc = jnp.zeros((L,), jnp.float32)
          for kk in range(EMB // L):
            sl = pl.ds(kk * L, L)
            d = hb[i, sl] + rb[i, sl] - tb[i, sl]
            acc = acc + d * d
          scr[j, pl.ds(0, L)] = acc
        svec = jnp.zeros((L,), jnp.float32)
        for l in range(L):
          col = jnp.full((L,), l, jnp.int32)
          svec = svec + plsc.load_gather(scr, [iota16, col])
        outv[pl.ds(pl.multiple_of(c * CH + g * L, L), L)] = svec
        return 0

      lax.fori_loop(0, CH // L, group, 0)

    pltpu.sync_copy(outv, out_hbm.at[pl.ds(wid * BPW, BPW)])

  return k


_score_kernel = _build()


def kernel(h, r, t, ent_emb, rel_emb):
  h2 = h.astype(jnp.int32).reshape(BATCH // CH, CH)
  r2 = r.astype(jnp.int32).reshape(BATCH // CH, CH)
  t2 = t.astype(jnp.int32).reshape(BATCH // CH, CH)
  return _score_kernel(h2, r2, t2, ent_emb, rel_emb)
